# baseline (speedup 1.0000x reference)
import jax
import jax.numpy as jnp
from jax.experimental import pallas as pl
from jax.experimental.pallas import tpu as pltpu

_N = 16384
_D = 512
_BLK = 4096
_GRID = _N // _BLK + 1


def _concat_body(idx_ref, x_ref, w_ref, o_ref):
    i = pl.program_id(0)

    @pl.when(i < _GRID - 1)
    def _copy():
        o_ref[...] = x_ref[...]

    @pl.when(i == _GRID - 1)
    def _tail():
        o_ref[pl.ds(0, 8), :] = jnp.broadcast_to(w_ref[0, 0], (8, _D))


def kernel(input_features, part_cls, obj_cls, W):
    idx = jnp.stack(
        [jnp.asarray(part_cls, jnp.int32), jnp.asarray(obj_cls, jnp.int32)]
    )
    grid_spec = pltpu.PrefetchScalarGridSpec(
        num_scalar_prefetch=1,
        grid=(_GRID,),
        in_specs=[
            pl.BlockSpec(
                (_BLK, _D), lambda i, idx: (jnp.minimum(i, _N // _BLK - 1), 0)
            ),
            pl.BlockSpec((1, 1, 1, _D), lambda i, idx: (idx[0], idx[1], 0, 0)),
        ],
        out_specs=pl.BlockSpec((_BLK, _D), lambda i, idx: (i, 0)),
    )
    return pl.pallas_call(
        _concat_body,
        grid_spec=grid_spec,
        out_shape=jax.ShapeDtypeStruct((_N + 1, _D), jnp.float32),
    )(idx, input_features, W)
